# baseline (speedup 1.0000x reference)
import functools

import jax
import jax.numpy as jnp
from jax import lax
from jax.experimental import pallas as pl
from jax.experimental.pallas import tpu as pltpu
from jax.experimental.pallas import tpu_sc as plsc

VOCAB = 1000000
EMBED_DIM = 32
BATCH = 16384
HIST = 50
B = BATCH * HIST

NUM_CORES = 2
NUM_SUBCORES = 16
NW = NUM_CORES * NUM_SUBCORES
JL = 4
B_PER_W = B // NW
OUT_FLAT = BATCH * HIST * EMBED_DIM
HG = 5
NQ = HIST // HG

_mesh = plsc.VectorSubcoreMesh(core_axis_name="c", subcore_axis_name="s")


@functools.partial(
    pl.kernel,
    out_type=jax.ShapeDtypeStruct((OUT_FLAT,), jnp.float32),
    mesh=_mesh,
    scratch_types=[
        pltpu.VMEM((B_PER_W,), jnp.int32),
        pltpu.VMEM((B_PER_W,), jnp.int32),
        [pltpu.VMEM((HG * 128, EMBED_DIM), jnp.float32) for _ in range(2)],
        [pltpu.VMEM((8 * 128 * 4,), jnp.float32) for _ in range(2)],
        [pltpu.SemaphoreType.DMA for _ in range(2)],
        [pltpu.SemaphoreType.DMA for _ in range(2)],
    ],
    compiler_params=pltpu.CompilerParams(
        use_tc_tiling_on_sc=False, needs_layout_passes=False
    ),
)
def _gather_kernel(idx_hbm, table_hbm, out_hbm, xbuf, idx_t, rows, tr,
                   sem_g, sem_s):
    wid = lax.axis_index("s") * NUM_CORES + lax.axis_index("c")

    iota16 = lax.iota(jnp.int32, 16)
    pre = [(iota16 + 16 * m) * HIST for m in range(8)]
    colbase = [(iota16 + 16 * m) * EMBED_DIM for m in range(8)]

    pltpu.sync_copy(idx_hbm.at[pl.ds(wid * B_PER_W, B_PER_W)], xbuf)

    @plsc.parallel_loop(0, HIST, unroll=2)
    def idx_body(h):
        for jl in range(JL):
            base = jl * (128 * HIST) + h
            vs = [plsc.load_gather(xbuf, [pre[m] + base]) for m in range(8)]
            for m in range(8):
                idx_t[pl.ds((jl * HIST + h) * 128 + 16 * m, 16)] = vs[m]

    def fire_gather(c, s):
        pltpu.async_copy(
            table_hbm.at[idx_t.at[pl.ds(c * 128, HG * 128)]], rows[s], sem_g[s]
        )

    def wait_gather(s):
        pltpu.make_async_copy(
            table_hbm.at[idx_t.at[pl.ds(0, HG * 128)]], rows[s], sem_g[s]
        ).wait()

    rowids = [iota16 + 16 * m for m in range(8)]

    def transpose(s, hh2, t):
        @plsc.parallel_loop(0, EMBED_DIM, unroll=8)
        def t_body(d):
            dcol = jnp.full((16,), 0, jnp.int32) + d
            vs = [
                plsc.load_gather(rows[s], [rowids[m] + hh2 * 128, dcol])
                for m in range(8)
            ]
            for m in range(8):
                tr[t][pl.ds(d * 128 + 16 * m, 16)] = vs[m]

    def fire_stores(jg, h, t):
        for i in range(4):
            pltpu.async_copy(
                tr[t].at[pl.ds(i * 1024, 1024)],
                out_hbm.at[pl.ds((h * 4 + i) * 131072 + jg * 1024, 1024)],
                sem_s[t],
            )

    def wait_stores(t):
        for _ in range(4):
            pltpu.make_async_copy(
                tr[t].at[pl.ds(0, 1024)],
                out_hbm.at[pl.ds(0, 1024)],
                sem_s[t],
            ).wait()

    def jl_body(jl, carry):
        jg = wid * JL + jl
        c0 = jl * HIST
        fire_gather(c0, 0)

        def q_group(qq, carry2):
            for par in range(2):
                q = 2 * qq + par
                s = par

                @pl.when(q <= NQ - 2)
                def _():
                    fire_gather(c0 + (q + 1) * HG, 1 - s)

                wait_gather(s)

                for hh2 in range(HG):
                    t = (HG * par + hh2) % 2
                    sb = NQ * HG * 0 + q * HG + hh2

                    @pl.when(sb >= 2)
                    def _():
                        wait_stores(t)

                    transpose(s, hh2, t)
                    fire_stores(jg, q * HG + hh2, t)
            return carry2

        lax.fori_loop(0, NQ // 2, q_group, 0, unroll=False)
        wait_stores(0)
        wait_stores(1)
        return carry

    lax.fori_loop(0, JL, jl_body, 0, unroll=False)


def kernel(x, table):
    idx = x.reshape(-1).astype(jnp.int32)
    flat = _gather_kernel(idx, table)
    o = flat.reshape(HIST, 4, 128, 8, 128)
    o = o.transpose(2, 4, 0, 1, 3)
    return o.reshape(BATCH, HIST, EMBED_DIM)


# speedup vs baseline: 1.0011x; 1.0011x over previous
import functools

import jax
import jax.numpy as jnp
from jax import lax
from jax.experimental import pallas as pl
from jax.experimental.pallas import tpu as pltpu
from jax.experimental.pallas import tpu_sc as plsc

VOCAB = 1000000
EMBED_DIM = 32
BATCH = 16384
HIST = 50
B = BATCH * HIST

NUM_CORES = 2
NUM_SUBCORES = 16
NW = NUM_CORES * NUM_SUBCORES
JL = 4
B_PER_W = B // NW
OUT_FLAT = BATCH * HIST * EMBED_DIM
HG = 2
NQ = HIST // HG

_mesh = plsc.VectorSubcoreMesh(core_axis_name="c", subcore_axis_name="s")


@functools.partial(
    pl.kernel,
    out_type=jax.ShapeDtypeStruct((OUT_FLAT,), jnp.float32),
    mesh=_mesh,
    scratch_types=[
        pltpu.VMEM((B_PER_W,), jnp.int32),
        pltpu.VMEM((B_PER_W,), jnp.int32),
        [pltpu.VMEM((HG * 128, 4 * EMBED_DIM), jnp.float32) for _ in range(2)],
        [pltpu.VMEM((8 * 128 * 4,), jnp.float32) for _ in range(2)],
        [pltpu.SemaphoreType.DMA for _ in range(2)],
        [pltpu.SemaphoreType.DMA for _ in range(2)],
    ],
    compiler_params=pltpu.CompilerParams(
        use_tc_tiling_on_sc=False, needs_layout_passes=False
    ),
)
def _gather_kernel(idx_hbm, t128_hbm, out_hbm, xbuf, idx_t, rows, tr,
                   sem_g, sem_s):
    wid = lax.axis_index("s") * NUM_CORES + lax.axis_index("c")

    iota16 = lax.iota(jnp.int32, 16)
    pre = [(iota16 + 16 * m) * HIST for m in range(8)]
    rowids = [iota16 + 16 * m for m in range(8)]

    pltpu.sync_copy(idx_hbm.at[pl.ds(wid * B_PER_W, B_PER_W)], xbuf)

    @plsc.parallel_loop(0, HIST, unroll=2)
    def idx_body(h):
        for jl in range(JL):
            base = jl * (128 * HIST) + h
            vs = [plsc.load_gather(xbuf, [pre[m] + base]) for m in range(8)]
            for m in range(8):
                idx_t[pl.ds((jl * HIST + h) * 128 + 16 * m, 16)] = vs[m]

    @plsc.parallel_loop(0, B_PER_W // 16, unroll=4)
    def div_body(i):
        v = idx_t[pl.ds(i * 16, 16)]
        xbuf[pl.ds(i * 16, 16)] = jnp.right_shift(v, 2)

    def fire_gather(c, s):
        pltpu.async_copy(
            t128_hbm.at[xbuf.at[pl.ds(c * 128, HG * 128)]], rows[s], sem_g[s]
        )

    def wait_gather(s):
        pltpu.make_async_copy(
            t128_hbm.at[xbuf.at[pl.ds(0, HG * 128)]], rows[s], sem_g[s]
        ).wait()

    def transpose(s, hh2, t, cg):
        basev = jnp.full((16,), 0, jnp.int32) + cg * 128
        qv = [plsc.load_gather(idx_t, [rowids[m] + basev]) for m in range(8)]
        qoff = [jnp.left_shift(jnp.bitwise_and(q, 3), 5) for q in qv]

        @plsc.parallel_loop(0, EMBED_DIM, unroll=4)
        def t_body(d):
            dcol = jnp.full((16,), 0, jnp.int32) + d
            vs = [
                plsc.load_gather(
                    rows[s], [rowids[m] + hh2 * 128, qoff[m] + dcol]
                )
                for m in range(8)
            ]
            for m in range(8):
                tr[t][pl.ds(d * 128 + 16 * m, 16)] = vs[m]

    def fire_stores(jg, h, t):
        for i in range(4):
            pltpu.async_copy(
                tr[t].at[pl.ds(i * 1024, 1024)],
                out_hbm.at[pl.ds((h * 4 + i) * 131072 + jg * 1024, 1024)],
                sem_s[t],
            )

    def wait_stores(t):
        for _ in range(4):
            pltpu.make_async_copy(
                tr[t].at[pl.ds(0, 1024)],
                out_hbm.at[pl.ds(0, 1024)],
                sem_s[t],
            ).wait()

    def process(q, s, jg, c0, with_store_waits):
        for hh2 in range(HG):
            t = hh2
            if with_store_waits:
                wait_stores(t)
            h = 2 * q + hh2
            transpose(s, hh2, t, c0 + h)
            fire_stores(jg, h, t)

    def jl_body(jl, carry):
        jg = wid * JL + jl
        c0 = jl * HIST
        fire_gather(c0, 0)
        fire_gather(c0 + HG, 1)
        wait_gather(0)
        process(0, 0, jg, c0, False)

        def q_group(qq, carry2):
            for par in range(2):
                q = 1 + 2 * qq + par
                s = 1 - par

                @pl.when(q <= NQ - 2)
                def _():
                    fire_gather(c0 + (q + 1) * HG, 1 - s)

                wait_gather(s)
                process(q, s, jg, c0, True)
            return carry2

        lax.fori_loop(0, (NQ - 1) // 2, q_group, 0, unroll=False)
        wait_stores(0)
        wait_stores(1)
        return carry

    lax.fori_loop(0, JL, jl_body, 0, unroll=False)


def kernel(x, table):
    idx = x.reshape(-1).astype(jnp.int32)
    t128 = table.reshape(VOCAB // 4, 4 * EMBED_DIM)
    flat = _gather_kernel(idx, t128)
    o = flat.reshape(HIST, 4, 128, 8, 128)
    o = o.transpose(2, 4, 0, 1, 3)
    return o.reshape(BATCH, HIST, EMBED_DIM)


# speedup vs baseline: 1.0265x; 1.0253x over previous
import functools

import jax
import jax.numpy as jnp
from jax import lax
from jax.experimental import pallas as pl
from jax.experimental.pallas import tpu as pltpu
from jax.experimental.pallas import tpu_sc as plsc

VOCAB = 1000000
EMBED_DIM = 32
BATCH = 16384
HIST = 50
B = BATCH * HIST

NUM_CORES = 2
NUM_SUBCORES = 16
NW = NUM_CORES * NUM_SUBCORES
JL = 4
B_PER_W = B // NW
OUT_FLAT = BATCH * HIST * EMBED_DIM
HG = 5
NQ = HIST // HG

_mesh = plsc.VectorSubcoreMesh(core_axis_name="c", subcore_axis_name="s")


@functools.partial(
    pl.kernel,
    out_type=jax.ShapeDtypeStruct((OUT_FLAT,), jnp.float32),
    mesh=_mesh,
    scratch_types=[
        pltpu.VMEM((B_PER_W,), jnp.int32),
        pltpu.VMEM((B_PER_W,), jnp.int32),
        [pltpu.VMEM((HG * 128, EMBED_DIM), jnp.float32) for _ in range(2)],
        [pltpu.VMEM((8 * 128 * 4,), jnp.float32) for _ in range(2)],
        [pltpu.SemaphoreType.DMA for _ in range(2)],
        [pltpu.SemaphoreType.DMA for _ in range(2)],
    ],
    compiler_params=pltpu.CompilerParams(
        use_tc_tiling_on_sc=False, needs_layout_passes=False
    ),
)
def _gather_kernel(idx_hbm, table_hbm, out_hbm, xbuf, idx_t, rows, tr,
                   sem_g, sem_s):
    wid = lax.axis_index("s") * NUM_CORES + lax.axis_index("c")

    iota16 = lax.iota(jnp.int32, 16)
    pre = [(iota16 + 16 * m) * HIST for m in range(8)]
    colbase = [(iota16 + 16 * m) * EMBED_DIM for m in range(8)]

    pltpu.sync_copy(idx_hbm.at[pl.ds(wid * B_PER_W, B_PER_W)], xbuf)

    @plsc.parallel_loop(0, HIST, unroll=2)
    def idx_body(h):
        for jl in range(JL):
            base = jl * (128 * HIST) + h
            vs = [plsc.load_gather(xbuf, [pre[m] + base]) for m in range(8)]
            for m in range(8):
                idx_t[pl.ds((jl * HIST + h) * 128 + 16 * m, 16)] = vs[m]

    def fire_gather(c, s):
        pltpu.async_copy(
            table_hbm.at[idx_t.at[pl.ds(c * 128, HG * 128)]], rows[s], sem_g[s]
        )

    def wait_gather(s):
        pltpu.make_async_copy(
            table_hbm.at[idx_t.at[pl.ds(0, HG * 128)]], rows[s], sem_g[s]
        ).wait()

    rowids = [iota16 + 16 * m for m in range(8)]

    def transpose(s, hh2, t):
        @plsc.parallel_loop(0, EMBED_DIM, unroll=4)
        def t_body(d):
            dcol = jnp.full((16,), 0, jnp.int32) + d
            vs = [
                plsc.load_gather(rows[s], [rowids[m] + hh2 * 128, dcol])
                for m in range(8)
            ]
            for m in range(8):
                tr[t][pl.ds(d * 128 + 16 * m, 16)] = vs[m]

    def fire_stores(jg, h, t):
        for i in range(4):
            pltpu.async_copy(
                tr[t].at[pl.ds(i * 1024, 1024)],
                out_hbm.at[pl.ds((h * 4 + i) * 131072 + jg * 1024, 1024)],
                sem_s[t],
            )

    def wait_stores(t):
        for _ in range(4):
            pltpu.make_async_copy(
                tr[t].at[pl.ds(0, 1024)],
                out_hbm.at[pl.ds(0, 1024)],
                sem_s[t],
            ).wait()

    def jl_body(jl, carry):
        jg = wid * JL + jl
        c0 = jl * HIST
        fire_gather(c0, 0)

        def q_group(qq, carry2):
            for par in range(2):
                q = 2 * qq + par
                s = par

                @pl.when(q <= NQ - 2)
                def _():
                    fire_gather(c0 + (q + 1) * HG, 1 - s)

                wait_gather(s)

                for hh2 in range(HG):
                    t = (HG * par + hh2) % 2
                    sb = NQ * HG * 0 + q * HG + hh2

                    @pl.when(sb >= 2)
                    def _():
                        wait_stores(t)

                    transpose(s, hh2, t)
                    fire_stores(jg, q * HG + hh2, t)
            return carry2

        lax.fori_loop(0, NQ // 2, q_group, 0, unroll=False)
        wait_stores(0)
        wait_stores(1)
        return carry

    lax.fori_loop(0, JL, jl_body, 0, unroll=False)


def kernel(x, table):
    idx = x.reshape(-1).astype(jnp.int32)
    flat = _gather_kernel(idx, table)
    o = flat.reshape(HIST, 4, 128, 8, 128)
    o = o.transpose(2, 4, 0, 1, 3)
    return o.reshape(BATCH, HIST, EMBED_DIM)
